# baseline (speedup 1.0000x reference)
import jax
import jax.numpy as jnp
from jax.experimental import pallas as pl
from jax.experimental.pallas import tpu as pltpu


def _make_body(Y, inv_n, n_steps):
    def body(out_ref, acc_ref):
        i = pl.program_id(0)
        y = [out_ref[0, 0, t] for t in range(Y)]
        z = [out_ref[0, 1, t] for t in range(Y)]
        zero = jnp.zeros_like(y[0])

        best = jnp.full_like(y[0], -7.0)
        f = zero
        for t in range(2, Y - 1):
            d = y[t] - y[t - 1]
            take = d < best
            best = jnp.where(take, d, best)
            f = jnp.where(take, jnp.float32(t), f)

        SyB = zero
        SxyB = zero
        Ty = zero
        Tty = zero
        for t in range(Y):
            yt = y[t]
            ty = t * yt if t else zero
            Ty = Ty + yt
            Tty = Tty + ty
            if t <= Y - 3:
                m = f > t
                SyB = SyB + jnp.where(m, yt, 0.0)
                SxyB = SxyB + jnp.where(m, ty, 0.0)

        nB = jnp.maximum(f, 2.0)
        mxB = (nB - 1.0) * 0.5
        varB = nB * (nB * nB - 1.0) * (1.0 / 12.0)
        slopeB = (SxyB - mxB * SyB) / varB
        cB = jnp.clip(SyB / nB - slopeB * mxB, 0.0, 100.0)
        sB = jnp.clip(slopeB, 0.0, 2.0)

        nA = Y - f
        SyA = Ty - SyB
        SxyA = (Tty - SxyB) - f * SyA
        mxA = (nA - 1.0) * 0.5
        varA = nA * (nA * nA - 1.0) * (1.0 / 12.0)
        slopeA = (SxyA - mxA * SyA) / varA
        cA = jnp.clip(SyA / nA - slopeA * mxA, 0.0, 100.0)
        sA = jnp.clip(slopeA, 0.0, 2.0)

        acc = zero
        for t in range(Y):
            if t <= Y - 3:
                isB = f > t
                s = jnp.where(isB, sB, sA)
                c = jnp.where(isB, cB, cA)
                xl = jnp.where(isB, jnp.float32(t), t - f)
            else:
                s, c, xl = sA, cA, t - f
            r = s * xl + c - z[t]
            acc = acc + r * r

        partial = jnp.sum(acc) * inv_n

        @pl.when(i == 0)
        def _init():
            acc_ref[0, 0] = 0.0

        acc_ref[0, 0] += partial

    return body


def kernel(out, target):
    del target
    B, _, Y, H, W = out.shape
    BH = 32
    nH = H // BH
    n_steps = B * nH
    body = _make_body(Y, 1.0 / (B * Y * H * W), n_steps)
    res = pl.pallas_call(
        body,
        grid=(n_steps,),
        in_specs=[
            pl.BlockSpec((1, 2, Y, BH, W), lambda i: (i // nH, 0, 0, i % nH, 0))
        ],
        out_specs=pl.BlockSpec((1, 1), lambda i: (0, 0),
                               memory_space=pltpu.SMEM),
        out_shape=jax.ShapeDtypeStruct((1, 1), jnp.float32),
    )(out)
    return res[0, 0]


# speedup vs baseline: 1.3229x; 1.3229x over previous
import jax
import jax.numpy as jnp
from jax.experimental import pallas as pl
from jax.experimental.pallas import tpu as pltpu


def _chunk_loss(y, z, Y):
    zero = jnp.zeros_like(y[0])

    best = jnp.full_like(y[0], -7.0)
    f = zero
    for t in range(2, Y - 1):
        d = y[t] - y[t - 1]
        take = d < best
        best = jnp.where(take, d, best)
        f = jnp.where(take, jnp.float32(t), f)

    m = [f > t for t in range(Y - 2)]

    ty = [t * y[t] for t in range(1, Y)]
    Ty = y[0]
    for t in range(1, Y):
        Ty = Ty + y[t]
    Tty = ty[0]
    for t in range(2, Y):
        Tty = Tty + ty[t - 1]
    SyB = jnp.where(m[0], y[0], 0.0)
    for t in range(1, Y - 2):
        SyB = SyB + jnp.where(m[t], y[t], 0.0)
    SxyB = jnp.where(m[1], ty[0], 0.0)
    for t in range(2, Y - 2):
        SxyB = SxyB + jnp.where(m[t], ty[t - 1], 0.0)

    nB = jnp.maximum(f, 2.0)
    mxB = (nB - 1.0) * 0.5
    varB = nB * (nB * nB - 1.0) * (1.0 / 12.0)
    slopeB = (SxyB - mxB * SyB) / varB
    cB = jnp.clip(SyB / nB - slopeB * mxB, 0.0, 100.0)
    sB = jnp.clip(slopeB, 0.0, 2.0)

    nA = Y - f
    SyA = Ty - SyB
    SxyA = (Tty - SxyB) - f * SyA
    mxA = (nA - 1.0) * 0.5
    varA = nA * (nA * nA - 1.0) * (1.0 / 12.0)
    slopeA = (SxyA - mxA * SyA) / varA
    cA = jnp.clip(SyA / nA - slopeA * mxA, 0.0, 100.0)
    sA = jnp.clip(slopeA, 0.0, 2.0)
    cAg = cA - sA * f

    acc = zero
    for t in range(Y):
        if t == 0:
            r = jnp.where(m[0], cB, cAg) - z[0]
        elif t <= Y - 3:
            s = jnp.where(m[t], sB, sA)
            c = jnp.where(m[t], cB, cAg)
            r = s * t + c - z[t]
        else:
            r = sA * t + cAg - z[t]
        acc = acc + r * r
    return acc


def _make_body(Y, BH, W, inv_n):
    CR = 32

    def body(out_ref, acc_ref):
        i = pl.program_id(0)

        def chunk(k, acc):
            r = k * CR
            y = [out_ref[0, 0, t, pl.ds(r, CR), :] for t in range(Y)]
            z = [out_ref[0, 1, t, pl.ds(r, CR), :] for t in range(Y)]
            return acc + _chunk_loss(y, z, Y)

        acc = jax.lax.fori_loop(
            0, BH // CR, chunk, jnp.zeros((CR, W), jnp.float32)
        )
        partial = jnp.sum(acc) * inv_n

        @pl.when(i == 0)
        def _init():
            acc_ref[0, 0] = 0.0

        acc_ref[0, 0] += partial

    return body


def kernel(out, target):
    del target
    B, _, Y, H, W = out.shape
    BH = 64
    nH = H // BH
    n_steps = B * nH
    body = _make_body(Y, BH, W, 1.0 / (B * Y * H * W))
    res = pl.pallas_call(
        body,
        grid=(n_steps,),
        in_specs=[
            pl.BlockSpec((1, 2, Y, BH, W), lambda i: (i // nH, 0, 0, i % nH, 0))
        ],
        out_specs=pl.BlockSpec((1, 1), lambda i: (0, 0),
                               memory_space=pltpu.SMEM),
        out_shape=jax.ShapeDtypeStruct((1, 1), jnp.float32),
    )(out)
    return res[0, 0]


# speedup vs baseline: 1.5447x; 1.1677x over previous
import jax
import jax.numpy as jnp
from jax.experimental import pallas as pl
from jax.experimental.pallas import tpu as pltpu


def _chunk_loss(y, z, Y):
    zero = jnp.zeros_like(y[0])

    best = jnp.full_like(y[0], -7.0)
    f = zero
    for t in range(2, Y - 1):
        d = y[t] - y[t - 1]
        take = d < best
        best = jnp.where(take, d, best)
        f = jnp.where(take, jnp.float32(t), f)

    m = [f > t for t in range(Y - 2)]

    ty = [t * y[t] for t in range(1, Y)]
    Ty = y[0]
    for t in range(1, Y):
        Ty = Ty + y[t]
    Tty = ty[0]
    for t in range(2, Y):
        Tty = Tty + ty[t - 1]
    SyB = jnp.where(m[0], y[0], 0.0)
    for t in range(1, Y - 2):
        SyB = SyB + jnp.where(m[t], y[t], 0.0)
    SxyB = jnp.where(m[1], ty[0], 0.0)
    for t in range(2, Y - 2):
        SxyB = SxyB + jnp.where(m[t], ty[t - 1], 0.0)

    nB = jnp.maximum(f, 2.0)
    mxB = (nB - 1.0) * 0.5
    varB = nB * (nB * nB - 1.0) * (1.0 / 12.0)
    slopeB = (SxyB - mxB * SyB) / varB
    cB = jnp.clip(SyB / nB - slopeB * mxB, 0.0, 100.0)
    sB = jnp.clip(slopeB, 0.0, 2.0)

    nA = Y - f
    SyA = Ty - SyB
    SxyA = (Tty - SxyB) - f * SyA
    mxA = (nA - 1.0) * 0.5
    varA = nA * (nA * nA - 1.0) * (1.0 / 12.0)
    slopeA = (SxyA - mxA * SyA) / varA
    cA = jnp.clip(SyA / nA - slopeA * mxA, 0.0, 100.0)
    sA = jnp.clip(slopeA, 0.0, 2.0)
    cAg = cA - sA * f

    acc = zero
    for t in range(Y):
        if t == 0:
            r = jnp.where(m[0], cB, cAg) - z[0]
        elif t <= Y - 3:
            s = jnp.where(m[t], sB, sA)
            c = jnp.where(m[t], cB, cAg)
            r = s * t + c - z[t]
        else:
            r = sA * t + cAg - z[t]
        acc = acc + r * r
    return acc


def _make_body(Y, BH, W, inv_n):
    CR = 32

    def body(out_ref, acc_ref):
        i = pl.program_id(0)

        def chunk(k, acc):
            r = k * CR
            y = [out_ref[0, 0, t, pl.ds(r, CR), :] for t in range(Y)]
            z = [out_ref[0, 1, t, pl.ds(r, CR), :] for t in range(Y)]
            return acc + _chunk_loss(y, z, Y)

        acc = jax.lax.fori_loop(
            0, BH // CR, chunk, jnp.zeros((CR, W), jnp.float32)
        )
        partial = jnp.sum(acc) * inv_n

        @pl.when(i == 0)
        def _init():
            acc_ref[0, 0] = 0.0

        acc_ref[0, 0] += partial

    return body


def kernel(out, target):
    del target
    B, _, Y, H, W = out.shape
    BH = 128
    nH = H // BH
    n_steps = B * nH
    body = _make_body(Y, BH, W, 1.0 / (B * Y * H * W))
    res = pl.pallas_call(
        body,
        grid=(n_steps,),
        in_specs=[
            pl.BlockSpec((1, 2, Y, BH, W), lambda i: (i // nH, 0, 0, i % nH, 0))
        ],
        out_specs=pl.BlockSpec((1, 1), lambda i: (0, 0),
                               memory_space=pltpu.SMEM),
        out_shape=jax.ShapeDtypeStruct((1, 1), jnp.float32),
    )(out)
    return res[0, 0]


# speedup vs baseline: 1.5583x; 1.0088x over previous
import jax
import jax.numpy as jnp
from jax.experimental import pallas as pl
from jax.experimental.pallas import tpu as pltpu


def _chunk_loss(y, z, Y):
    zero = jnp.zeros_like(y[0])

    best = jnp.full_like(y[0], -7.0)
    f = zero
    for t in range(2, Y - 1):
        d = y[t] - y[t - 1]
        take = d < best
        best = jnp.where(take, d, best)
        f = jnp.where(take, jnp.float32(t), f)

    m = [f > t for t in range(Y - 2)]

    ty = [t * y[t] for t in range(1, Y)]
    Ty = y[0]
    for t in range(1, Y):
        Ty = Ty + y[t]
    Tty = ty[0]
    for t in range(2, Y):
        Tty = Tty + ty[t - 1]
    SyB = jnp.where(m[0], y[0], 0.0)
    for t in range(1, Y - 2):
        SyB = SyB + jnp.where(m[t], y[t], 0.0)
    SxyB = jnp.where(m[1], ty[0], 0.0)
    for t in range(2, Y - 2):
        SxyB = SxyB + jnp.where(m[t], ty[t - 1], 0.0)

    nB = jnp.maximum(f, 2.0)
    mxB = (nB - 1.0) * 0.5
    varB = nB * (nB * nB - 1.0) * (1.0 / 12.0)
    slopeB = (SxyB - mxB * SyB) / varB
    cB = jnp.clip(SyB / nB - slopeB * mxB, 0.0, 100.0)
    sB = jnp.clip(slopeB, 0.0, 2.0)

    nA = Y - f
    SyA = Ty - SyB
    SxyA = (Tty - SxyB) - f * SyA
    mxA = (nA - 1.0) * 0.5
    varA = nA * (nA * nA - 1.0) * (1.0 / 12.0)
    slopeA = (SxyA - mxA * SyA) / varA
    cA = jnp.clip(SyA / nA - slopeA * mxA, 0.0, 100.0)
    sA = jnp.clip(slopeA, 0.0, 2.0)
    cAg = cA - sA * f

    acc = zero
    for t in range(Y):
        if t == 0:
            r = jnp.where(m[0], cB, cAg) - z[0]
        elif t <= Y - 3:
            s = jnp.where(m[t], sB, sA)
            c = jnp.where(m[t], cB, cAg)
            r = s * t + c - z[t]
        else:
            r = sA * t + cAg - z[t]
        acc = acc + r * r
    return acc


def _make_body(Y, BH, W, inv_n):
    CR = 32

    def body(out_ref, acc_ref):
        i = pl.program_id(0)

        def chunk(k, acc):
            r = k * CR
            y = [out_ref[0, 0, t, pl.ds(r, CR), :] for t in range(Y)]
            z = [out_ref[0, 1, t, pl.ds(r, CR), :] for t in range(Y)]
            return acc + _chunk_loss(y, z, Y)

        acc = jax.lax.fori_loop(
            0, BH // CR, chunk, jnp.zeros((CR, W), jnp.float32)
        )
        partial = jnp.sum(acc) * inv_n

        @pl.when(i == 0)
        def _init():
            acc_ref[0, 0] = 0.0

        acc_ref[0, 0] += partial

    return body


def kernel(out, target):
    del target
    B, _, Y, H, W = out.shape
    BH = 256
    nH = H // BH
    n_steps = B * nH
    body = _make_body(Y, BH, W, 1.0 / (B * Y * H * W))
    res = pl.pallas_call(
        body,
        grid=(n_steps,),
        in_specs=[
            pl.BlockSpec((1, 2, Y, BH, W), lambda i: (i // nH, 0, 0, i % nH, 0))
        ],
        out_specs=pl.BlockSpec((1, 1), lambda i: (0, 0),
                               memory_space=pltpu.SMEM),
        out_shape=jax.ShapeDtypeStruct((1, 1), jnp.float32),
    )(out)
    return res[0, 0]
